# baseline (speedup 1.0000x reference)
import functools

import jax
import jax.numpy as jnp
from jax import lax
from jax.experimental import pallas as pl
from jax.experimental.pallas import tpu as pltpu
from jax.experimental.pallas import tpu_sc as plsc

_N = 3_200_000
_D = 16
_NW = 16
_B_PER_W = _N // _NW
_CH = 2_000
_NCH = _B_PER_W // _CH

_mesh = plsc.VectorSubcoreMesh(core_axis_name="c", subcore_axis_name="s", num_cores=1)


@functools.partial(
    pl.kernel,
    out_type=jax.ShapeDtypeStruct((_N * _D,), jnp.float32),
    mesh=_mesh,
    compiler_params=pltpu.CompilerParams(use_tc_tiling_on_sc=False,
                                         needs_layout_passes=False),
    scratch_types=[
        pltpu.VMEM((_D * 10,), jnp.float32),
        [pltpu.VMEM((_CH,), jnp.int32)] * 2,
        [pltpu.VMEM((_CH * _D,), jnp.float32)] * 2,
        [pltpu.SemaphoreType.DMA] * 2,
        [pltpu.SemaphoreType.DMA] * 2,
    ],
)
def _emb_lookup(idx_hbm, table_hbm, out_hbm, tbl_v, idx_v, rows_v, isem, osem):
    wid = lax.axis_index("s")
    base = wid * _B_PER_W
    pltpu.sync_copy(table_hbm, tbl_v)
    iota16t = lax.iota(jnp.int32, 16) * _D

    def idx_copy(c, b):
        return pltpu.make_async_copy(
            idx_hbm.at[pl.ds(base + c * _CH, _CH)], idx_v[b], isem[b])

    def out_copy(c, b):
        return pltpu.make_async_copy(
            rows_v[b], out_hbm.at[pl.ds((base + c * _CH) * _D, _CH * _D)],
            osem[b])

    def expand(b):
        @plsc.parallel_loop(0, _CH // 16, unroll=5)
        def blk(k):
            idx_vec = idx_v[b][pl.ds(k * 16, 16)]
            fidx = idx_vec * _D
            sbase = iota16t + k * (16 * _D)
            for j in range(_D):
                col = plsc.load_gather(tbl_v, [fidx + j])
                plsc.store_scatter(rows_v[b], [sbase + j], col)

    idx_copy(0, 0).start()
    idx_copy(1, 1).start()
    for b in range(2):
        idx_copy(b, b).wait()
        expand(b)
        out_copy(b, b).start()
        idx_copy(b + 2, b).start()

    def pair_body(p, carry):
        c0 = 2 * p
        for b in range(2):
            c = c0 + b
            idx_copy(c, b).wait()
            out_copy(c - 2, b).wait()
            expand(b)
            out_copy(c, b).start()
            idx_copy(c + 2, b).start()
        return carry

    lax.fori_loop(1, _NCH // 2 - 1, pair_body, 0)

    for b in range(2):
        c = _NCH - 2 + b
        idx_copy(c, b).wait()
        out_copy(c - 2, b).wait()
        expand(b)
        out_copy(c, b).start()
    out_copy(_NCH - 2, 0).wait()
    out_copy(_NCH - 1, 1).wait()


def kernel(bond_types, embedding_weight):
    flat = _emb_lookup(bond_types.astype(jnp.int32),
                       embedding_weight.reshape(-1))
    return flat.reshape(_N, _D)


# speedup vs baseline: 1.7552x; 1.7552x over previous
import functools

import jax
import jax.numpy as jnp
from jax import lax
from jax.experimental import pallas as pl
from jax.experimental.pallas import tpu as pltpu
from jax.experimental.pallas import tpu_sc as plsc

_N = 3_200_000
_D = 16
_R = 10
_NW = 32
_B_PER_W = _N // _NW
_CH = 2_000
_NCH = _B_PER_W // _CH

_mesh = plsc.VectorSubcoreMesh(core_axis_name="c", subcore_axis_name="s")


@functools.partial(
    pl.kernel,
    out_type=jax.ShapeDtypeStruct((_N * _D,), jnp.float32),
    mesh=_mesh,
    compiler_params=pltpu.CompilerParams(use_tc_tiling_on_sc=False,
                                         needs_layout_passes=False),
    scratch_types=[
        pltpu.VMEM((_R * _D * 16,), jnp.float32),
        [pltpu.VMEM((_CH,), jnp.int32)] * 2,
        [pltpu.VMEM((_CH * _D,), jnp.float32)] * 2,
        [pltpu.SemaphoreType.DMA] * 2,
        [pltpu.SemaphoreType.DMA] * 2,
    ],
)
def _emb_lookup(idx_hbm, table_hbm, out_hbm, tbl_v, idx_v, rows_v, isem, osem):
    wid = lax.axis_index("s") * 2 + lax.axis_index("c")
    base = wid * _B_PER_W
    pltpu.sync_copy(table_hbm, tbl_v)
    iota17 = lax.iota(jnp.int32, 16) * (_D + 1)

    def idx_copy(c, b):
        return pltpu.make_async_copy(
            idx_hbm.at[pl.ds(base + c * _CH, _CH)], idx_v[b], isem[b])

    def out_copy(c, b):
        return pltpu.make_async_copy(
            rows_v[b], out_hbm.at[pl.ds((base + c * _CH) * _D, _CH * _D)],
            osem[b])

    def expand(b):
        @plsc.parallel_loop(0, _CH // 16, unroll=5)
        def blk(k):
            idx_vec = idx_v[b][pl.ds(k * 16, 16)]
            sidx = idx_vec * (_D * 16)
            for e in range(16):
                splat = lax.gather(
                    sidx, jnp.full((16, 1), e, jnp.int32),
                    lax.GatherDimensionNumbers(offset_dims=(),
                                               collapsed_slice_dims=(0,),
                                               start_index_map=(0,)),
                    (1,), mode=lax.GatherScatterMode.PROMISE_IN_BOUNDS)
                row = plsc.load_gather(tbl_v, [splat + iota17])
                rows_v[b][pl.ds(k * 256 + e * 16, 16)] = row

    idx_copy(0, 0).start()
    idx_copy(1, 1).start()
    for b in range(2):
        idx_copy(b, b).wait()
        expand(b)
        out_copy(b, b).start()
        idx_copy(b + 2, b).start()

    def pair_body(p, carry):
        c0 = 2 * p
        for b in range(2):
            c = c0 + b
            idx_copy(c, b).wait()
            out_copy(c - 2, b).wait()
            expand(b)
            out_copy(c, b).start()
            idx_copy(c + 2, b).start()
        return carry

    lax.fori_loop(1, _NCH // 2 - 1, pair_body, 0)

    for b in range(2):
        c = _NCH - 2 + b
        idx_copy(c, b).wait()
        out_copy(c - 2, b).wait()
        expand(b)
        out_copy(c, b).start()
    out_copy(_NCH - 2, 0).wait()
    out_copy(_NCH - 1, 1).wait()


def kernel(bond_types, embedding_weight):
    rep = jnp.broadcast_to(embedding_weight[:, :, None],
                           (_R, _D, 16)).reshape(-1)
    flat = _emb_lookup(bond_types.astype(jnp.int32), rep)
    return flat.reshape(_N, _D)


# speedup vs baseline: 14.7350x; 8.3951x over previous
import functools

import jax
import jax.numpy as jnp
from jax import lax
from jax.experimental import pallas as pl
from jax.experimental.pallas import tpu as pltpu
from jax.experimental.pallas import tpu_sc as plsc

_N = 3_200_000
_D = 16
_R = 10
_NW = 32
_CH = 2_560
_T = _CH // 128
_G = _N // _CH

_mesh = plsc.VectorSubcoreMesh(core_axis_name="c", subcore_axis_name="s")


@functools.partial(
    pl.kernel,
    out_type=jax.ShapeDtypeStruct((2, _N // 128, 8, 128), jnp.float32),
    mesh=_mesh,
    compiler_params=pltpu.CompilerParams(use_tc_tiling_on_sc=False,
                                         needs_layout_passes=False),
    scratch_types=[
        pltpu.VMEM((_R * _D * 16,), jnp.float32),
        pltpu.VMEM((_CH,), jnp.int32),
        pltpu.VMEM((2, _T, 8, 128), jnp.float32),
    ],
)
def _emb_lookup(idx_hbm, table_hbm, out_hbm, tbl_v, idx_v, buf_v):
    wid = lax.axis_index("s") * 2 + lax.axis_index("c")
    pltpu.sync_copy(table_hbm, tbl_v)
    iota16 = lax.iota(jnp.int32, 16)
    trips = jnp.where(wid < _G - (_G // _NW) * _NW, _G // _NW + 1, _G // _NW)

    def chunk_body(i, carry):
        g = wid + _NW * i
        pltpu.sync_copy(idx_hbm.at[pl.ds(g * _CH, _CH)], idx_v)

        @plsc.parallel_loop(0, _CH // 16, unroll=4)
        def blk(k):
            idx_vec = idx_v[pl.ds(k * 16, 16)]
            gbase = idx_vec * (_D * 16) + iota16
            t = k // 8
            e0 = (k % 8) * 16
            for j in range(_D):
                col = plsc.load_gather(tbl_v, [gbase + j * 16])
                buf_v[j // 8, t, j % 8, pl.ds(e0, 16)] = col

        for p in range(2):
            pltpu.sync_copy(buf_v.at[p],
                            out_hbm.at[p, pl.ds(g * _T, _T)])
        return carry

    lax.fori_loop(0, trips, chunk_body, 0)


def kernel(bond_types, embedding_weight):
    rep = jnp.broadcast_to(embedding_weight[:, :, None],
                           (_R, _D, 16)).reshape(-1)
    tiles = _emb_lookup(bond_types.astype(jnp.int32), rep)
    return jnp.transpose(tiles, (1, 3, 0, 2)).reshape(_N, _D)


# speedup vs baseline: 25.8615x; 1.7551x over previous
import functools

import jax
import jax.numpy as jnp
from jax import lax
from jax.experimental import pallas as pl
from jax.experimental.pallas import tpu as pltpu
from jax.experimental.pallas import tpu_sc as plsc

_N = 3_200_000
_D = 16
_R = 10
_NW = 32
_CH = 2_560
_T = _CH // 128
_G = _N // _CH

_mesh = plsc.VectorSubcoreMesh(core_axis_name="c", subcore_axis_name="s")


@functools.partial(
    pl.kernel,
    out_type=jax.ShapeDtypeStruct((2, _N // 128, 8, 128), jnp.float32),
    mesh=_mesh,
    compiler_params=pltpu.CompilerParams(use_tc_tiling_on_sc=False,
                                         needs_layout_passes=False),
    scratch_types=[
        pltpu.VMEM((_R * _D * 16,), jnp.float32),
        [pltpu.VMEM((_CH,), jnp.int32)] * 2,
        [pltpu.VMEM((2, _T, 8, 128), jnp.float32)] * 2,
        [pltpu.SemaphoreType.DMA] * 2,
        [pltpu.SemaphoreType.DMA] * 2,
    ],
)
def _emb_lookup(idx_hbm, table_hbm, out_hbm, tbl_v, idx_v, buf_v, isem, osem):
    wid = lax.axis_index("s") * 2 + lax.axis_index("c")
    pltpu.sync_copy(table_hbm, tbl_v)
    iota16 = lax.iota(jnp.int32, 16)
    trips = jnp.where(wid < _G - (_G // _NW) * _NW, _G // _NW + 1, _G // _NW)

    def chunk_of(i):
        return jnp.minimum(wid + _NW * i, _G - 1)

    def idx_start(i, b):
        pltpu.make_async_copy(idx_hbm.at[pl.ds(chunk_of(i) * _CH, _CH)],
                              idx_v[b], isem[b]).start()

    def idx_wait(i, b):
        pltpu.make_async_copy(idx_hbm.at[pl.ds(chunk_of(i) * _CH, _CH)],
                              idx_v[b], isem[b]).wait()

    def out_copies(i, b):
        g = chunk_of(i)
        return [pltpu.make_async_copy(buf_v[b].at[p],
                                      out_hbm.at[p, pl.ds(g * _T, _T)],
                                      osem[b]) for p in range(2)]

    def out_start(i, b):
        for cp in out_copies(i, b):
            cp.start()

    def out_wait(i, b):
        for cp in out_copies(i, b):
            cp.wait()

    def expand(b):
        @plsc.parallel_loop(0, _CH // 16, unroll=4)
        def blk(k):
            idx_vec = idx_v[b][pl.ds(k * 16, 16)]
            gbase = idx_vec * (_D * 16) + iota16
            t = k // 8
            e0 = (k % 8) * 16
            for j in range(_D):
                col = plsc.load_gather(tbl_v, [gbase + j * 16])
                buf_v[b][j // 8, t, j % 8, pl.ds(e0, 16)] = col

    idx_start(0, 0)
    idx_start(1, 1)
    for b in range(2):
        idx_wait(b, b)
        expand(b)
        out_start(b, b)
        idx_start(b + 2, b)

    def pair_body(pi, carry):
        a = 2 * pi
        for b in range(2):
            idx_wait(a + b, b)
            out_wait(a + b - 2, b)
            expand(b)
            out_start(a + b, b)
            idx_start(a + b + 2, b)
        return carry

    lax.fori_loop(1, trips // 2, pair_body, 0)

    last_even = 2 * (trips // 2)

    @pl.when(trips % 2 == 1)
    def _remainder():
        idx_wait(last_even, 0)
        out_wait(last_even - 2, 0)
        expand(0)
        out_start(last_even, 0)
        idx_wait(last_even + 1, 1)

    @pl.when(trips % 2 == 0)
    def _drain_idx():
        idx_wait(last_even, 0)
        idx_wait(last_even + 1, 1)

    out_wait(trips - 2, 0)
    out_wait(trips - 1, 1)


def kernel(bond_types, embedding_weight):
    rep = jnp.broadcast_to(embedding_weight[:, :, None],
                           (_R, _D, 16)).reshape(-1)
    tiles = _emb_lookup(bond_types.astype(jnp.int32), rep)
    return jnp.transpose(tiles, (1, 3, 0, 2)).reshape(_N, _D)
